# baseline (speedup 1.0000x reference)
import functools

import jax
import jax.numpy as jnp
from jax import lax
from jax.experimental import pallas as pl
from jax.experimental.pallas import tpu as pltpu
from jax.experimental.pallas import tpu_sc as plsc

_N = 10000
_E = 160000
_NC = 2
_NS = 16
_DC = 128
_EB = 50
_NB = _E // _NS // _EB
_EBD = 40
_NBD = _E // (_NC * _NS) // _EBD
_K = 5
_NP = 10240
_RPT = _NP // _NS
_BN = 2000


def _sc_mesh():
    return plsc.VectorSubcoreMesh(
        core_axis_name="c", subcore_axis_name="s",
        num_cores=_NC, num_subcores=_NS)


@functools.partial(
    pl.kernel,
    out_type=jax.ShapeDtypeStruct((_NC, _NP, _DC), jnp.float32),
    mesh=_sc_mesh(),
    scratch_types=[
        pltpu.VMEM((_NBD, _EBD), jnp.int32),
        pltpu.VMEM((_EBD, _DC), jnp.float32),
        pltpu.VMEM_SHARED((_NP, _DC), jnp.float32),
        pltpu.SemaphoreType.DMA,
    ],
)
def _deg_kernel(dst_hbm, zeros_hbm, ones_hbm, out_hbm, didx, ones, hist, sem):
    cid = lax.axis_index("c")
    sid = lax.axis_index("s")
    rbase = sid * _RPT
    pltpu.sync_copy(dst_hbm.at[cid].at[sid], didx)
    pltpu.sync_copy(ones_hbm, ones)
    pltpu.sync_copy(zeros_hbm.at[pl.ds(rbase, _RPT)],
                    hist.at[pl.ds(rbase, _RPT)])
    plsc.subcore_barrier()

    def group(g, carry):
        for k in range(_K):
            b = g * _K + k
            pltpu.sync_copy(ones, hist.at[didx.at[b]], add=True)
        return carry

    lax.fori_loop(0, _NBD // _K, group, 0)
    plsc.subcore_barrier()
    pltpu.sync_copy(hist.at[pl.ds(rbase, _RPT)],
                    out_hbm.at[cid].at[pl.ds(rbase, _RPT)])


def _make_agg(nchunks):
    npass = nchunks // _NC
    G = _K
    ngi = _NB // G

    @functools.partial(
        pl.kernel,
        out_type=jax.ShapeDtypeStruct((nchunks, _NP, _DC), jnp.float32),
        mesh=_sc_mesh(),
        scratch_types=[
            pltpu.VMEM((3 * G, _EB), jnp.int32),
            pltpu.VMEM((3 * G, _EB), jnp.int32),
            pltpu.VMEM((G, _EB, _DC), jnp.float32),
            pltpu.VMEM_SHARED((_NP, _DC), jnp.float32),
            pltpu.SemaphoreType.DMA,
            pltpu.SemaphoreType.DMA,
        ],
    )
    def agg(xs_hbm, src_hbm, dst_hbm, zeros_hbm, out_hbm,
            sidx, didx, rows, acc, sem_g, sem_i):
        cid = lax.axis_index("c")
        sid = lax.axis_index("s")
        rbase = sid * _RPT

        def fire_idx(j, slot3):
            pltpu.async_copy(src_hbm.at[sid].at[j],
                             sidx.at[pl.ds(slot3 * G, G)], sem_i)
            pltpu.async_copy(dst_hbm.at[sid].at[j],
                             didx.at[pl.ds(slot3 * G, G)], sem_i)

        def wait_idx():
            pltpu.make_async_copy(src_hbm.at[sid].at[0],
                                  sidx.at[pl.ds(0, G)], sem_i).wait()
            pltpu.make_async_copy(dst_hbm.at[sid].at[0],
                                  didx.at[pl.ds(0, G)], sem_i).wait()

        for p in range(npass):
            chunk = p * _NC + cid
            xs_c = xs_hbm.at[chunk]
            pltpu.sync_copy(zeros_hbm.at[pl.ds(rbase, _RPT)],
                            acc.at[pl.ds(rbase, _RPT)])
            fire_idx(0, 0)
            fire_idx(1, 1)
            fire_idx(2, 2)
            plsc.subcore_barrier()
            wait_idx()

            def prime(r, carry):
                pltpu.async_copy(xs_c.at[sidx.at[r]], rows.at[r], sem_g)
                return carry

            lax.fori_loop(0, G, prime, 0)

            def outer(gi, carry):
                wait_idx()
                base = (gi % 3) * G
                basen = ((gi + 1) % 3) * G

                def inner(r, c2):
                    pltpu.make_async_copy(
                        xs_c.at[sidx.at[base + r]], rows.at[r],
                        sem_g).wait()
                    pltpu.sync_copy(rows.at[r], acc.at[didx.at[base + r]],
                                    add=True)
                    pltpu.async_copy(
                        xs_c.at[sidx.at[basen + r]], rows.at[r], sem_g)
                    return c2

                lax.fori_loop(0, G, inner, 0)
                fire_idx(jnp.minimum(gi + 3, ngi - 1), gi % 3)
                return carry

            lax.fori_loop(0, ngi, outer, 0)

            wait_idx()
            wait_idx()

            def draing(r, carry):
                pltpu.make_async_copy(xs_c.at[sidx.at[r]], rows.at[r],
                                      sem_g).wait()
                return carry

            lax.fori_loop(0, G, draing, 0)
            plsc.subcore_barrier()
            pltpu.sync_copy(acc.at[pl.ds(rbase, _RPT)],
                            out_hbm.at[chunk].at[pl.ds(rbase, _RPT)])

    return agg


_agg2 = _make_agg(2)
_agg4 = _make_agg(4)


def _sdeg(deg_ref):
    d = deg_ref[0, :, 0:1] + deg_ref[1, :, 0:1] + 1.0
    return lax.rsqrt(d)


def _scale_body(deg_ref, x_ref, xs_ref):
    s = _sdeg(deg_ref)
    xs = x_ref[...] * s
    xs_ref[0] = xs[:, :_DC]
    xs_ref[1] = xs[:, _DC:]


def _tc_scale(degp, x):
    return pl.pallas_call(
        _scale_body,
        grid=(_N // _BN,),
        in_specs=[
            pl.BlockSpec((2, _BN, _DC), lambda i: (0, i, 0)),
            pl.BlockSpec((_BN, 256), lambda i: (i, 0)),
        ],
        out_specs=pl.BlockSpec((2, _BN, _DC), lambda i: (0, i, 0)),
        out_shape=jax.ShapeDtypeStruct((2, _N, _DC), jnp.float32),
    )(degp, x)


def _layer1_body(deg_ref, a_ref, xs_ref, w_ref, b_ref, out_ref):
    s = _sdeg(deg_ref)
    z = jnp.concatenate(
        [a_ref[0] + xs_ref[0], a_ref[1] + xs_ref[1]], axis=1) * s
    h = jnp.dot(z.astype(jnp.bfloat16), w_ref[...].astype(jnp.bfloat16),
                preferred_element_type=jnp.float32)
    hs = jnp.maximum(h + b_ref[...], 0.0) * s
    for c in range(4):
        out_ref[c] = hs[:, c * _DC:(c + 1) * _DC]


def _tc_layer1(degp, agg1, xs, W1, b1):
    return pl.pallas_call(
        _layer1_body,
        grid=(_N // _BN,),
        in_specs=[
            pl.BlockSpec((2, _BN, _DC), lambda i: (0, i, 0)),
            pl.BlockSpec((2, _BN, _DC), lambda i: (0, i, 0)),
            pl.BlockSpec((2, _BN, _DC), lambda i: (0, i, 0)),
            pl.BlockSpec((256, 512), lambda i: (0, 0)),
            pl.BlockSpec((1, 512), lambda i: (0, 0)),
        ],
        out_specs=pl.BlockSpec((4, _BN, _DC), lambda i: (0, i, 0)),
        out_shape=jax.ShapeDtypeStruct((4, _N, _DC), jnp.float32),
    )(degp, agg1, xs, W1, b1.reshape(1, -1))


def _layer2_body(deg_ref, a_ref, h_ref, w_ref, b_ref, out_ref):
    i = pl.program_id(0)
    s = _sdeg(deg_ref)
    z = jnp.concatenate(
        [a_ref[c] + h_ref[c] for c in range(4)], axis=1) * s
    p = jnp.dot(z.astype(jnp.bfloat16), w_ref[...].astype(jnp.bfloat16),
                preferred_element_type=jnp.float32)
    r = jnp.maximum(p + b_ref[...], 0.0)

    @pl.when(i == 0)
    def _():
        out_ref[...] = jnp.zeros_like(out_ref)

    out_ref[...] += jnp.sum(r, axis=0, keepdims=True)

    @pl.when(i == pl.num_programs(0) - 1)
    def _():
        out_ref[...] = out_ref[...] * (1.0 / _N)


def _tc_layer2(degp, agg2, h1s, W2, b2):
    return pl.pallas_call(
        _layer2_body,
        grid=(_N // _BN,),
        in_specs=[
            pl.BlockSpec((2, _BN, _DC), lambda i: (0, i, 0)),
            pl.BlockSpec((4, _BN, _DC), lambda i: (0, i, 0)),
            pl.BlockSpec((4, _BN, _DC), lambda i: (0, i, 0)),
            pl.BlockSpec((512, 1024), lambda i: (0, 0)),
            pl.BlockSpec((1, 1024), lambda i: (0, 0)),
        ],
        out_specs=pl.BlockSpec((1, 1024), lambda i: (0, 0)),
        out_shape=jax.ShapeDtypeStruct((1, 1024), jnp.float32),
    )(degp, agg2, h1s, W2, b2.reshape(1, -1))


def kernel(x, edge_index, W1, b1, W2, b2):
    src = edge_index[0].astype(jnp.int32)
    dst = edge_index[1].astype(jnp.int32)
    src_t = src.reshape(_NS, _NB // _K, _K, _EB)
    dst_t = dst.reshape(_NS, _NB // _K, _K, _EB)
    dst_d = dst.reshape(_NC, _NS, _NBD, _EBD)
    zeros128 = jnp.zeros((_NP, _DC), jnp.float32)
    ones128 = jnp.ones((_EBD, _DC), jnp.float32)

    degp = _deg_kernel(dst_d, zeros128, ones128)
    xs = _tc_scale(degp, x)
    agg1 = _agg2(xs, src_t, dst_t, zeros128)
    h1s = _tc_layer1(degp, agg1, xs, W1, b1)
    agg2 = _agg4(h1s, src_t, dst_t, zeros128)
    out = _tc_layer2(degp, agg2, h1s, W2, b2)
    return out.reshape(-1)


# speedup vs baseline: 1.0006x; 1.0006x over previous
import functools

import jax
import jax.numpy as jnp
from jax import lax
from jax.experimental import pallas as pl
from jax.experimental.pallas import tpu as pltpu
from jax.experimental.pallas import tpu_sc as plsc

_N = 10000
_E = 160000
_NC = 2
_NS = 16
_DC = 128
_EB = 50
_NB = _E // _NS // _EB
_EBD = 40
_NBD = _E // (_NC * _NS) // _EBD
_K = 5
_NP = 10240
_RPT = _NP // _NS
_BN = 2000


def _sc_mesh():
    return plsc.VectorSubcoreMesh(
        core_axis_name="c", subcore_axis_name="s",
        num_cores=_NC, num_subcores=_NS)


@functools.partial(
    pl.kernel,
    out_type=jax.ShapeDtypeStruct((_NC, _NP, _DC), jnp.float32),
    mesh=_sc_mesh(),
    scratch_types=[
        pltpu.VMEM((_NBD, _EBD), jnp.int32),
        pltpu.VMEM((_EBD, _DC), jnp.float32),
        pltpu.VMEM_SHARED((_NP, _DC), jnp.float32),
        pltpu.SemaphoreType.DMA,
    ],
)
def _deg_kernel(dst_hbm, zeros_hbm, ones_hbm, out_hbm, didx, ones, hist, sem):
    cid = lax.axis_index("c")
    sid = lax.axis_index("s")
    rbase = sid * _RPT
    pltpu.sync_copy(dst_hbm.at[cid].at[sid], didx)
    pltpu.sync_copy(ones_hbm, ones)
    pltpu.sync_copy(zeros_hbm.at[pl.ds(rbase, _RPT)],
                    hist.at[pl.ds(rbase, _RPT)])
    plsc.subcore_barrier()

    def group(g, carry):
        for k in range(_K):
            b = g * _K + k
            pltpu.sync_copy(ones, hist.at[didx.at[b]], add=True)
        return carry

    lax.fori_loop(0, _NBD // _K, group, 0)
    plsc.subcore_barrier()
    pltpu.sync_copy(hist.at[pl.ds(rbase, _RPT)],
                    out_hbm.at[cid].at[pl.ds(rbase, _RPT)])


def _make_agg(nchunks):
    npass = nchunks // _NC
    G = _K
    ngi = _NB // G

    @functools.partial(
        pl.kernel,
        out_type=jax.ShapeDtypeStruct((nchunks, _NP, _DC), jnp.float32),
        mesh=_sc_mesh(),
        scratch_types=[
            pltpu.VMEM((3 * G, _EB), jnp.int32),
            pltpu.VMEM((3 * G, _EB), jnp.int32),
            pltpu.VMEM((G, _EB, _DC), jnp.float32),
            pltpu.VMEM_SHARED((_NP, _DC), jnp.float32),
            pltpu.SemaphoreType.DMA,
            pltpu.SemaphoreType.DMA,
        ],
    )
    def agg(xs_hbm, src_hbm, dst_hbm, zeros_hbm, out_hbm,
            sidx, didx, rows, acc, sem_g, sem_i):
        cid = lax.axis_index("c")
        sid = lax.axis_index("s")
        rbase = sid * _RPT

        def fire_idx(j, slot3):
            pltpu.async_copy(src_hbm.at[sid].at[j],
                             sidx.at[pl.ds(slot3 * G, G)], sem_i)
            pltpu.async_copy(dst_hbm.at[sid].at[j],
                             didx.at[pl.ds(slot3 * G, G)], sem_i)

        def wait_idx():
            pltpu.make_async_copy(src_hbm.at[sid].at[0],
                                  sidx.at[pl.ds(0, G)], sem_i).wait()
            pltpu.make_async_copy(dst_hbm.at[sid].at[0],
                                  didx.at[pl.ds(0, G)], sem_i).wait()

        for p in range(npass):
            chunk = p * _NC + cid
            xs_c = xs_hbm.at[chunk]
            pltpu.sync_copy(zeros_hbm.at[pl.ds(rbase, _RPT)],
                            acc.at[pl.ds(rbase, _RPT)])
            fire_idx(0, 0)
            fire_idx(1, 1)
            fire_idx(2, 2)
            plsc.subcore_barrier()
            wait_idx()

            def prime(r, carry):
                pltpu.async_copy(xs_c.at[sidx.at[r]], rows.at[r], sem_g)
                return carry

            lax.fori_loop(0, G, prime, 0)

            def outer(gi, carry):
                wait_idx()
                base = (gi % 3) * G
                basen = ((gi + 1) % 3) * G

                def inner(r, c2):
                    pltpu.make_async_copy(
                        xs_c.at[sidx.at[base + r]], rows.at[r],
                        sem_g).wait()
                    pltpu.sync_copy(rows.at[r], acc.at[didx.at[base + r]],
                                    add=True)
                    pltpu.async_copy(
                        xs_c.at[sidx.at[basen + r]], rows.at[r], sem_g)
                    return c2

                lax.fori_loop(0, G, inner, 0)
                fire_idx(jnp.minimum(gi + 3, ngi - 1), gi % 3)
                return carry

            lax.fori_loop(0, ngi, outer, 0)

            wait_idx()
            wait_idx()

            def draing(r, carry):
                pltpu.make_async_copy(xs_c.at[sidx.at[r]], rows.at[r],
                                      sem_g).wait()
                return carry

            lax.fori_loop(0, G, draing, 0)
            plsc.subcore_barrier()
            pltpu.sync_copy(acc.at[pl.ds(rbase, _RPT)],
                            out_hbm.at[chunk].at[pl.ds(rbase, _RPT)])

    return agg


_agg2 = _make_agg(2)
_agg4 = _make_agg(4)


def _sdeg(deg_ref):
    d = deg_ref[0, :, 0:1] + deg_ref[1, :, 0:1] + 1.0
    return lax.rsqrt(d)


def _scale_body(deg_ref, x_ref, xs_ref):
    s = _sdeg(deg_ref)
    xs = x_ref[...] * s
    xs_ref[0] = xs[:, :_DC]
    xs_ref[1] = xs[:, _DC:]


def _tc_scale(degp, x):
    return pl.pallas_call(
        _scale_body,
        grid=(_N // _BN,),
        in_specs=[
            pl.BlockSpec((2, _BN, _DC), lambda i: (0, i, 0)),
            pl.BlockSpec((_BN, 256), lambda i: (i, 0)),
        ],
        out_specs=pl.BlockSpec((2, _BN, _DC), lambda i: (0, i, 0)),
        out_shape=jax.ShapeDtypeStruct((2, _N, _DC), jnp.float32),
    )(degp, x)


def _layer1_body(deg_ref, a_ref, xs_ref, w_ref, b_ref, out_ref):
    s = _sdeg(deg_ref)
    z = jnp.concatenate(
        [a_ref[0] + xs_ref[0], a_ref[1] + xs_ref[1]], axis=1) * s
    h = jnp.dot(z, w_ref[...], preferred_element_type=jnp.float32)
    hs = jnp.maximum(h + b_ref[...], 0.0) * s
    for c in range(4):
        out_ref[c] = hs[:, c * _DC:(c + 1) * _DC]


def _tc_layer1(degp, agg1, xs, W1, b1):
    return pl.pallas_call(
        _layer1_body,
        grid=(_N // _BN,),
        in_specs=[
            pl.BlockSpec((2, _BN, _DC), lambda i: (0, i, 0)),
            pl.BlockSpec((2, _BN, _DC), lambda i: (0, i, 0)),
            pl.BlockSpec((2, _BN, _DC), lambda i: (0, i, 0)),
            pl.BlockSpec((256, 512), lambda i: (0, 0)),
            pl.BlockSpec((1, 512), lambda i: (0, 0)),
        ],
        out_specs=pl.BlockSpec((4, _BN, _DC), lambda i: (0, i, 0)),
        out_shape=jax.ShapeDtypeStruct((4, _N, _DC), jnp.float32),
    )(degp, agg1, xs, W1, b1.reshape(1, -1))


def _layer2_body(deg_ref, a_ref, h_ref, w_ref, b_ref, out_ref):
    i = pl.program_id(0)
    s = _sdeg(deg_ref)
    z = jnp.concatenate(
        [a_ref[c] + h_ref[c] for c in range(4)], axis=1) * s
    p = jnp.dot(z, w_ref[...], preferred_element_type=jnp.float32)
    r = jnp.maximum(p + b_ref[...], 0.0)

    @pl.when(i == 0)
    def _():
        out_ref[...] = jnp.zeros_like(out_ref)

    out_ref[...] += jnp.sum(r, axis=0, keepdims=True)

    @pl.when(i == pl.num_programs(0) - 1)
    def _():
        out_ref[...] = out_ref[...] * (1.0 / _N)


def _tc_layer2(degp, agg2, h1s, W2, b2):
    return pl.pallas_call(
        _layer2_body,
        grid=(_N // _BN,),
        in_specs=[
            pl.BlockSpec((2, _BN, _DC), lambda i: (0, i, 0)),
            pl.BlockSpec((4, _BN, _DC), lambda i: (0, i, 0)),
            pl.BlockSpec((4, _BN, _DC), lambda i: (0, i, 0)),
            pl.BlockSpec((512, 1024), lambda i: (0, 0)),
            pl.BlockSpec((1, 1024), lambda i: (0, 0)),
        ],
        out_specs=pl.BlockSpec((1, 1024), lambda i: (0, 0)),
        out_shape=jax.ShapeDtypeStruct((1, 1024), jnp.float32),
    )(degp, agg2, h1s, W2, b2.reshape(1, -1))


def kernel(x, edge_index, W1, b1, W2, b2):
    src = edge_index[0].astype(jnp.int32)
    dst = edge_index[1].astype(jnp.int32)
    src_t = src.reshape(_NS, _NB // _K, _K, _EB)
    dst_t = dst.reshape(_NS, _NB // _K, _K, _EB)
    dst_d = dst.reshape(_NC, _NS, _NBD, _EBD)
    zeros128 = jnp.zeros((_NP, _DC), jnp.float32)
    ones128 = jnp.ones((_EBD, _DC), jnp.float32)

    degp = _deg_kernel(dst_d, zeros128, ones128)
    xs = _tc_scale(degp, x)
    agg1 = _agg2(xs, src_t, dst_t, zeros128)
    h1s = _tc_layer1(degp, agg1, xs, W1, b1)
    agg2 = _agg4(h1s, src_t, dst_t, zeros128)
    out = _tc_layer2(degp, agg2, h1s, W2, b2)
    return out.reshape(-1)
